# baseline (speedup 1.0000x reference)
import functools

import jax
import jax.numpy as jnp
import numpy as np
from jax import lax
from jax.experimental import pallas as pl
from jax.experimental.pallas import tpu as pltpu
from jax.experimental.pallas import tpu_sc as plsc

N = 10000
E = 320000
D = 128
NC = 2
NS = 16
NW = NC * NS
CHUNK = 128
EPT = 10240
NCHUNK = EPT // CHUNK
IRD = 4
E_PAD = EPT * NW
_PERM = np.zeros(D, np.int32)
for _g in range(D // 32):
    for _i in range(16):
        _PERM[_g * 32 + 2 * _i] = _g * 32 + _i
        _PERM[_g * 32 + 2 * _i + 1] = _g * 32 + 16 + _i
SLAB = 632
SLAB_LAST = N - SLAB * (NS - 1)
ACC_ROWS = N


def _mm_body(x_ref, w_ref, b_ref, o_ref):
    o_ref[...] = (
        lax.dot_general(
            x_ref[...], w_ref[...], (((1,), (1,)), ((), ())),
            preferred_element_type=jnp.float32,
        )
        + b_ref[...]
    ).astype(jnp.bfloat16)


def _linear(x, W, b2):
    rblk = 2000
    nr = N // rblk
    return pl.pallas_call(
        _mm_body,
        grid=(nr,),
        in_specs=[
            pl.BlockSpec((rblk, D), lambda r: (r, 0)),
            pl.BlockSpec((D, D), lambda r: (0, 0)),
            pl.BlockSpec((1, D), lambda r: (0, 0)),
        ],
        out_specs=pl.BlockSpec((rblk, D), lambda r: (r, 0)),
        out_shape=jax.ShapeDtypeStruct((N, D), jnp.bfloat16),
    )(x, W, b2)


def _add_body(a_ref, b_ref, o_ref):
    o_ref[...] = a_ref[...] + b_ref[...]


def _combine(p):
    rblk = 2000
    nr = N // rblk
    return pl.pallas_call(
        _add_body,
        grid=(nr,),
        in_specs=[
            pl.BlockSpec((rblk, D), lambda r: (r, 0)),
            pl.BlockSpec((rblk, D), lambda r: (r + N // rblk, 0)),
        ],
        out_specs=pl.BlockSpec((rblk, D), lambda r: (r, 0)),
        out_shape=jax.ShapeDtypeStruct((N, D), jnp.float32),
    )(p, p)


def _sc_body(table, edata, out, iring, gbufs, pbufs, rbuf, accum, gsem,
             ssem, isem):
    core = lax.axis_index("c")
    sid = lax.axis_index("s")
    wid = core * NS + sid
    ebase = wid * NCHUNK

    zero16 = jnp.zeros((16,), jnp.float32)

    def _zrow(i, _):
        for f in range(D // 16):
            pbufs[0, i, pl.ds(f * 16, 16)] = zero16
        return _

    lax.fori_loop(0, CHUNK, _zrow, None, unroll=2)
    zbase = pl.multiple_of(sid * SLAB, 8)

    @pl.when(sid < NS - 1)
    def _zfull():
        for off in range(0, SLAB, CHUNK):
            n = min(CHUNK, SLAB - off)
            pltpu.sync_copy(pbufs.at[0, pl.ds(0, n)],
                            accum.at[pl.ds(pl.multiple_of(zbase + off, 8), n)])

    @pl.when(sid == NS - 1)
    def _zlast():
        for off in range(0, SLAB_LAST, CHUNK):
            n = min(CHUNK, SLAB_LAST - off)
            pltpu.sync_copy(pbufs.at[0, pl.ds(0, n)],
                            accum.at[pl.ds(pl.multiple_of(zbase + off, 8), n)])

    plsc.subcore_barrier()

    def _stage(j, s):
        pltpu.async_copy(edata.at[ebase + j], iring.at[pl.ds(3 * s, 3)],
                         isem.at[s])

    def _gather(j, s, b):
        pltpu.make_async_copy(edata.at[ebase + j], iring.at[pl.ds(3 * s, 3)],
                              isem.at[s]).wait()
        pltpu.async_copy(table.at[iring.at[3 * s]], gbufs.at[b], gsem.at[b])

    for s in range(IRD):
        _stage(s, s)
    _gather(0, 0, 0)
    _gather(1, 1, 1)

    def _chunkgrp(jg, _):
        for u in range(IRD):
            b = u % 2
            s = u
            j = jg * IRD + u
            pltpu.make_async_copy(table.at[iring.at[3 * s]], gbufs.at[b],
                                  gsem.at[b]).wait()

            @pl.when(j >= 2)
            def _drain():
                pltpu.make_async_copy(pbufs.at[b], accum.at[rbuf.at[b]],
                                      ssem.at[b]).wait()

            def _edge(e, _c):
                sv = jnp.full((16,), 3 * s + 2, jnp.int32)
                ev = jnp.full((16,), e, jnp.int32)
                splat = plsc.bitcast(
                    plsc.load_gather(iring, [sv, ev]), jnp.float32)
                for g in range(D // 32):
                    v = plsc.bitcast(gbufs[b, e, pl.ds(g * 16, 16)],
                                     jnp.bfloat16)
                    av, bv = plsc.unpack(v, format=plsc.PackFormat.INTERLEAVED)
                    pbufs[b, e, pl.ds(g * 32, 16)] = av * splat
                    pbufs[b, e, pl.ds(g * 32 + 16, 16)] = bv * splat
                return _c

            lax.fori_loop(0, CHUNK, _edge, None, unroll=8)
            for f in range(CHUNK // 16):
                rbuf[b, pl.ds(f * 16, 16)] = iring[3 * s + 1,
                                                   pl.ds(f * 16, 16)]
            pltpu.async_copy(pbufs.at[b], accum.at[rbuf.at[b]],
                             ssem.at[b], add=True)

            @pl.when(j + IRD < NCHUNK)
            def _restage():
                _stage(j + IRD, s)

            @pl.when(j + 2 < NCHUNK)
            def _next():
                _gather(j + 2, (s + 2) % IRD, b)
        return _

    lax.fori_loop(0, NCHUNK // IRD, _chunkgrp, None)
    for b in range(2):
        pltpu.make_async_copy(pbufs.at[b], accum.at[rbuf.at[b]],
                              ssem.at[b]).wait()
    plsc.subcore_barrier()

    src = pl.multiple_of(sid * SLAB, 8)
    dst = pl.multiple_of(core * N + sid * SLAB, 8)

    @pl.when(sid < NS - 1)
    def _full():
        pltpu.sync_copy(accum.at[pl.ds(src, SLAB)], out.at[pl.ds(dst, SLAB)])

    @pl.when(sid == NS - 1)
    def _last():
        pltpu.sync_copy(accum.at[pl.ds(src, SLAB_LAST)],
                        out.at[pl.ds(dst, SLAB_LAST)])


@functools.cache
def _sc_aggregate():
    @functools.partial(
        pl.kernel,
        out_type=jax.ShapeDtypeStruct((NC * N, D), jnp.float32),
        mesh=plsc.VectorSubcoreMesh(core_axis_name="c", subcore_axis_name="s",
                                    num_cores=NC, num_subcores=NS),
        compiler_params=pltpu.CompilerParams(needs_layout_passes=False,
                                             use_tc_tiling_on_sc=False),
        scratch_types=[
            pltpu.VMEM((IRD * 3, CHUNK), jnp.int32),
            pltpu.VMEM((2, CHUNK, D // 2), jnp.int32),
            pltpu.VMEM((2, CHUNK, D), jnp.float32),
            pltpu.VMEM((2, CHUNK), jnp.int32),
            pltpu.VMEM_SHARED((ACC_ROWS, D), jnp.float32),
            pltpu.SemaphoreType.DMA((2,)),
            pltpu.SemaphoreType.DMA((2,)),
            pltpu.SemaphoreType.DMA((IRD,)),
        ],
    )
    def agg(table, edata, out, *scratch):
        _sc_body(table, edata, out, *scratch)

    return agg


def kernel(input_features, adj_edge_index, adj_values, W, b):
    table_bf16 = _linear(input_features, W[_PERM], b[_PERM].reshape(1, D))
    table = lax.bitcast_convert_type(
        table_bf16.reshape(N, D // 2, 2), jnp.int32)

    col = adj_edge_index[1]
    row = adj_edge_index[0]
    pad = E_PAD - E
    col_p = jnp.pad(col, (0, pad))
    row_p = jnp.pad(row, (0, pad))
    val_p = jnp.pad(adj_values, (0, pad))
    valbits = lax.bitcast_convert_type(val_p, jnp.int32)
    edata = jnp.stack(
        [col_p.reshape(NW, NCHUNK, CHUNK),
         row_p.reshape(NW, NCHUNK, CHUNK),
         valbits.reshape(NW, NCHUNK, CHUNK)], axis=2,
    ).reshape(NW * NCHUNK, 3, CHUNK)

    partials = _sc_aggregate()(table, edata)
    return _combine(partials)


# speedup vs baseline: 1.0822x; 1.0822x over previous
import functools

import jax
import jax.numpy as jnp
import numpy as np
from jax import lax
from jax.experimental import pallas as pl
from jax.experimental.pallas import tpu as pltpu
from jax.experimental.pallas import tpu_sc as plsc

N = 10000
E = 320000
D = 128
NC = 2
NS = 16
NW = NC * NS
CHUNK = 128
EPT = 10240
NCHUNK = EPT // CHUNK
IRD = 4
E_PAD = EPT * NW
_PERM = np.zeros(D, np.int32)
for _g in range(D // 32):
    for _i in range(16):
        _PERM[_g * 32 + 2 * _i] = _g * 32 + _i
        _PERM[_g * 32 + 2 * _i + 1] = _g * 32 + 16 + _i
SLAB = 632
SLAB_LAST = N - SLAB * (NS - 1)
ACC_ROWS = N


def _mm_body(x_ref, w_ref, b_ref, o_ref):
    o_ref[...] = (
        lax.dot_general(
            x_ref[...], w_ref[...], (((1,), (1,)), ((), ())),
            preferred_element_type=jnp.float32,
        )
        + b_ref[...]
    ).astype(jnp.bfloat16)


def _linear(x, W, b2):
    rblk = 2000
    nr = N // rblk
    return pl.pallas_call(
        _mm_body,
        grid=(nr,),
        in_specs=[
            pl.BlockSpec((rblk, D), lambda r: (r, 0)),
            pl.BlockSpec((D, D), lambda r: (0, 0)),
            pl.BlockSpec((1, D), lambda r: (0, 0)),
        ],
        out_specs=pl.BlockSpec((rblk, D), lambda r: (r, 0)),
        out_shape=jax.ShapeDtypeStruct((N, D), jnp.bfloat16),
    )(x, W, b2)


def _add_body(a_ref, b_ref, o_ref):
    o_ref[...] = a_ref[...] + b_ref[...]


def _combine(p):
    rblk = 2000
    nr = N // rblk
    return pl.pallas_call(
        _add_body,
        grid=(nr,),
        in_specs=[
            pl.BlockSpec((rblk, D), lambda r: (r, 0)),
            pl.BlockSpec((rblk, D), lambda r: (r + N // rblk, 0)),
        ],
        out_specs=pl.BlockSpec((rblk, D), lambda r: (r, 0)),
        out_shape=jax.ShapeDtypeStruct((N, D), jnp.float32),
    )(p, p)


def _sc_body(table, edata, out, iring, gbufs, pbufs, rbuf, accum, gsem,
             ssem, isem):
    core = lax.axis_index("c")
    sid = lax.axis_index("s")
    wid = core * NS + sid
    ebase = wid * NCHUNK

    zero16 = jnp.zeros((16,), jnp.float32)

    def _zrow(i, _):
        for f in range(D // 16):
            pbufs[0, i, pl.ds(f * 16, 16)] = zero16
        return _

    lax.fori_loop(0, CHUNK, _zrow, None, unroll=2)
    zbase = pl.multiple_of(sid * SLAB, 8)

    @pl.when(sid < NS - 1)
    def _zfull():
        for off in range(0, SLAB, CHUNK):
            n = min(CHUNK, SLAB - off)
            pltpu.sync_copy(pbufs.at[0, pl.ds(0, n)],
                            accum.at[pl.ds(pl.multiple_of(zbase + off, 8), n)])

    @pl.when(sid == NS - 1)
    def _zlast():
        for off in range(0, SLAB_LAST, CHUNK):
            n = min(CHUNK, SLAB_LAST - off)
            pltpu.sync_copy(pbufs.at[0, pl.ds(0, n)],
                            accum.at[pl.ds(pl.multiple_of(zbase + off, 8), n)])

    plsc.subcore_barrier()

    def _stage(j, s):
        pltpu.async_copy(edata.at[ebase + j], iring.at[pl.ds(3 * s, 3)],
                         isem.at[s])

    def _gather(j, s, b):
        pltpu.make_async_copy(edata.at[ebase + j], iring.at[pl.ds(3 * s, 3)],
                              isem.at[s]).wait()
        pltpu.async_copy(table.at[iring.at[3 * s]], gbufs.at[b], gsem.at[b])

    for s in range(IRD):
        _stage(s, s)
    _gather(0, 0, 0)
    _gather(1, 1, 1)

    def _chunkgrp(jg, _):
        for u in range(IRD):
            b = u % 2
            s = u
            j = jg * IRD + u
            pltpu.make_async_copy(table.at[iring.at[3 * s]], gbufs.at[b],
                                  gsem.at[b]).wait()

            @pl.when(j >= 2)
            def _drain():
                pltpu.make_async_copy(pbufs.at[b], accum.at[rbuf.at[b]],
                                      ssem.at[b]).wait()

            def _edge(e, _c):
                sv = jnp.full((16,), 3 * s + 2, jnp.int32)
                ev = jnp.full((16,), e, jnp.int32)
                splat = plsc.bitcast(
                    plsc.load_gather(iring, [sv, ev]), jnp.float32)
                for g in range(D // 32):
                    v = plsc.bitcast(gbufs[b, e, pl.ds(g * 16, 16)],
                                     jnp.bfloat16)
                    av, bv = plsc.unpack(v, format=plsc.PackFormat.INTERLEAVED)
                    pbufs[b, e, pl.ds(g * 32, 16)] = av * splat
                    pbufs[b, e, pl.ds(g * 32 + 16, 16)] = bv * splat
                return _c

            lax.fori_loop(0, CHUNK, _edge, None, unroll=8)
            for f in range(CHUNK // 16):
                rbuf[b, pl.ds(f * 16, 16)] = iring[3 * s + 1,
                                                   pl.ds(f * 16, 16)]
            pltpu.async_copy(pbufs.at[b], accum.at[rbuf.at[b]],
                             ssem.at[b], add=True)

            @pl.when(j + IRD < NCHUNK)
            def _restage():
                _stage(j + IRD, s)

            @pl.when(j + 2 < NCHUNK)
            def _next():
                _gather(j + 2, (s + 2) % IRD, b)
        return _

    lax.fori_loop(0, NCHUNK // IRD, _chunkgrp, None)
    for b in range(2):
        pltpu.make_async_copy(pbufs.at[b], accum.at[rbuf.at[b]],
                              ssem.at[b]).wait()
    plsc.subcore_barrier()

    src = pl.multiple_of(sid * SLAB, 8)
    dst = pl.multiple_of(core * N + sid * SLAB, 8)

    @pl.when(sid < NS - 1)
    def _full():
        pltpu.sync_copy(accum.at[pl.ds(src, SLAB)], out.at[pl.ds(dst, SLAB)])

    @pl.when(sid == NS - 1)
    def _last():
        pltpu.sync_copy(accum.at[pl.ds(src, SLAB_LAST)],
                        out.at[pl.ds(dst, SLAB_LAST)])


@functools.cache
def _sc_aggregate():
    @functools.partial(
        pl.kernel,
        out_type=jax.ShapeDtypeStruct((NC * N, D), jnp.float32),
        mesh=plsc.VectorSubcoreMesh(core_axis_name="c", subcore_axis_name="s",
                                    num_cores=NC, num_subcores=NS),
        compiler_params=pltpu.CompilerParams(needs_layout_passes=False,
                                             use_tc_tiling_on_sc=False),
        scratch_types=[
            pltpu.VMEM((IRD * 3, CHUNK), jnp.int32),
            pltpu.VMEM((2, CHUNK, D // 2), jnp.int32),
            pltpu.VMEM((2, CHUNK, D), jnp.float32),
            pltpu.VMEM((2, CHUNK), jnp.int32),
            pltpu.VMEM_SHARED((ACC_ROWS, D), jnp.float32),
            pltpu.SemaphoreType.DMA((2,)),
            pltpu.SemaphoreType.DMA((2,)),
            pltpu.SemaphoreType.DMA((IRD,)),
        ],
    )
    def agg(table, edata, out, *scratch):
        _sc_body(table, edata, out, *scratch)

    return agg


def kernel(input_features, adj_edge_index, adj_values, W, b):
    table_bf16 = _linear(input_features, W[_PERM], b[_PERM].reshape(1, D))
    table = lax.bitcast_convert_type(
        table_bf16.reshape(N, D // 2, 2), jnp.int32)

    col = adj_edge_index[1]
    row = adj_edge_index[0]
    pad = E_PAD - E
    spread = jnp.arange(pad, dtype=jnp.int32) * 37 % N
    col_p = jnp.concatenate([col, spread])
    row_p = jnp.concatenate([row, spread])
    val_p = jnp.pad(adj_values, (0, pad))
    valbits = lax.bitcast_convert_type(val_p, jnp.int32)
    edata = jnp.stack(
        [col_p.reshape(NW, NCHUNK, CHUNK),
         row_p.reshape(NW, NCHUNK, CHUNK),
         valbits.reshape(NW, NCHUNK, CHUNK)], axis=2,
    ).reshape(NW * NCHUNK, 3, CHUNK)

    partials = _sc_aggregate()(table, edata)
    return _combine(partials)
